# baseline (speedup 1.0000x reference)
import functools

import jax
import jax.numpy as jnp
from jax import lax
from jax.experimental import pallas as pl
from jax.experimental.pallas import tpu as pltpu
from jax.experimental.pallas import tpu_sc as plsc

N = 10000
E = 320000
D = 128
H = 64
NC = 2
NS = 16
CHUNK = 64
NCHUNKS = E // CHUNK
BASE_CH = NCHUNKS // NS
EXTRA_CH = NCHUNKS % NS
ZROWS = 48
ZREP = 13
TAIL0 = NS * ZREP * ZROWS
TAILN = N - TAIL0
F32 = jnp.float32


def _dot(x, w):
    return lax.dot_general(x, w, (((1,), (1,)), ((), ())),
                           preferred_element_type=F32)


def _proj_body(h_ref, aw, ab, bw, bb, dw, db, ew, eb,
               ah_ref, dhbh_lo_ref, dhbh_hi_ref, eh_ref):
    hb = h_ref[...]
    ah = _dot(hb, aw[...]) + ab[...]
    bh = _dot(hb, bw[...]) + bb[...]
    dh = _dot(hb, dw[...]) + db[...]
    eh = _dot(hb, ew[...]) + eb[...]
    ah_ref[...] = ah
    dhbh_lo_ref[:, :H] = dh[:, :H]
    dhbh_lo_ref[:, H:] = bh[:, :H]
    dhbh_hi_ref[:, :H] = dh[:, H:]
    dhbh_hi_ref[:, H:] = bh[:, H:]
    eh_ref[...] = eh


def _run_proj(h, aw, ab, bw, bb, dw, db, ew, eb):
    nb = 10
    rb = N // nb
    wspec = pl.BlockSpec((D, D), lambda i: (0, 0))
    bspec = pl.BlockSpec((1, D), lambda i: (0, 0))
    return pl.pallas_call(
        _proj_body,
        grid=(nb,),
        in_specs=[pl.BlockSpec((rb, D), lambda i: (i, 0)),
                  wspec, bspec, wspec, bspec, wspec, bspec, wspec, bspec],
        out_specs=[pl.BlockSpec((rb, D), lambda i: (i, 0)),
                   pl.BlockSpec((rb, D), lambda i: (i, 0)),
                   pl.BlockSpec((rb, D), lambda i: (i, 0)),
                   pl.BlockSpec((rb, D), lambda i: (i, 0))],
        out_shape=[jax.ShapeDtypeStruct((N, D), F32),
                   jax.ShapeDtypeStruct((N, D), F32),
                   jax.ShapeDtypeStruct((N, D), F32),
                   jax.ShapeDtypeStruct((N, D), F32)],
    )(h, aw, ab, bw, bb, dw, db, ew, eb)


def _ce_body(e_ref, cw, cb, ce_lo_ref, ce_hi_ref):
    ce = _dot(e_ref[...], cw[...]) + cb[...]
    ce_lo_ref[...] = ce[:, :H]
    ce_hi_ref[...] = ce[:, H:]


def _run_ce(e, cw, cb):
    nb = 160
    rb = E // nb
    return pl.pallas_call(
        _ce_body,
        grid=(nb,),
        in_specs=[pl.BlockSpec((rb, D), lambda i: (i, 0)),
                  pl.BlockSpec((D, D), lambda i: (0, 0)),
                  pl.BlockSpec((1, D), lambda i: (0, 0))],
        out_specs=[pl.BlockSpec((rb, H), lambda i: (i, 0)),
                   pl.BlockSpec((rb, H), lambda i: (i, 0))],
        out_shape=[jax.ShapeDtypeStruct((E, H), F32),
                   jax.ShapeDtypeStruct((E, H), F32)],
    )(e, cw, cb)


def _sc_body(src_hbm, dst_hbm, dhbh_lo, dhbh_hi, eh_t, ce_lo, ce_hi,
             eij_lo, eij_hi, acc0, acc1, stats,
             src_v, dst_v, dhbh_v, eh_v, ce_v, eij_v, consig_v,
             zbuf, stats_v, acc_sh, sem):
    c = lax.axis_index("c")
    s = lax.axis_index("s")

    def _zb(i, carry):
        for k in range(D // 16):
            zbuf[i, pl.ds(k * 16, 16)] = jnp.zeros((16,), F32)
        return carry
    lax.fori_loop(0, ZROWS, _zb, 0)
    for j in range(ZREP):
        r0 = s * (ZREP * ZROWS) + j * ZROWS
        pltpu.sync_copy(zbuf, acc_sh.at[pl.ds(r0, ZROWS)])

    @pl.when(s == 0)
    def _():
        pltpu.sync_copy(zbuf.at[pl.ds(0, TAILN)], acc_sh.at[pl.ds(TAIL0, TAILN)])

    plsc.subcore_barrier()

    def chunk_body(i, carry):
        cid = s + i * NS
        base = cid * CHUNK
        pltpu.sync_copy(src_hbm.at[pl.ds(base, CHUNK)], src_v)
        pltpu.sync_copy(dst_hbm.at[pl.ds(base, CHUNK)], dst_v)

        d2 = pltpu.async_copy(eh_t.at[dst_v], eh_v, sem)

        @pl.when(c == 0)
        def _():
            d1 = pltpu.async_copy(dhbh_lo.at[src_v], dhbh_v, sem)
            d3 = pltpu.async_copy(ce_lo.at[pl.ds(base, CHUNK)], ce_v, sem)
            d1.wait(); d3.wait()

        @pl.when(c == 1)
        def _():
            d1 = pltpu.async_copy(dhbh_hi.at[src_v], dhbh_v, sem)
            d3 = pltpu.async_copy(ce_hi.at[pl.ds(base, CHUNK)], ce_v, sem)
            d1.wait(); d3.wait()

        d2.wait()

        def edge_body(ei, cc):
            accs = list(cc)
            for k in range(H // 16):
                sl = pl.ds(k * 16, 16)
                ehk = jnp.where(c == 0,
                                eh_v[ei, pl.ds(k * 16, 16)],
                                eh_v[ei, pl.ds(H + k * 16, 16)])
                x = ce_v[ei, sl] + dhbh_v[ei, sl] + ehk
                eij_v[ei, sl] = x
                sg = 1.0 / (1.0 + jnp.exp(-x))
                consig_v[ei, pl.ds(H + k * 16, 16)] = sg
                consig_v[ei, sl] = sg * dhbh_v[ei, pl.ds(H + k * 16, 16)]
                accs[k] = accs[k] + x
                accs[4 + k] = accs[4 + k] + x * x
            return tuple(accs)

        cc = lax.fori_loop(0, CHUNK, edge_body, carry)

        @pl.when(c == 0)
        def _():
            pltpu.sync_copy(eij_v, eij_lo.at[pl.ds(base, CHUNK)])

        @pl.when(c == 1)
        def _():
            pltpu.sync_copy(eij_v, eij_hi.at[pl.ds(base, CHUNK)])

        pltpu.sync_copy(consig_v, acc_sh.at[dst_v], add=True)
        return cc

    nch = BASE_CH + (s < EXTRA_CH).astype(jnp.int32)
    carry0 = tuple(jnp.zeros((16,), F32) for _ in range(8))
    carry = lax.fori_loop(0, nch, chunk_body, carry0)

    for k in range(H // 16):
        stats_v[0, pl.ds(k * 16, 16)] = carry[k]
        stats_v[1, pl.ds(k * 16, 16)] = carry[4 + k]
    pltpu.sync_copy(stats_v, stats.at[c, s])

    plsc.subcore_barrier()

    @pl.when(c == 0)
    def _():
        for j in range(ZREP):
            r0 = s * (ZREP * ZROWS) + j * ZROWS
            pltpu.sync_copy(acc_sh.at[pl.ds(r0, ZROWS)], acc0.at[pl.ds(r0, ZROWS)])

        @pl.when(s == 0)
        def _():
            pltpu.sync_copy(acc_sh.at[pl.ds(TAIL0, TAILN)], acc0.at[pl.ds(TAIL0, TAILN)])

    @pl.when(c == 1)
    def _():
        for j in range(ZREP):
            r0 = s * (ZREP * ZROWS) + j * ZROWS
            pltpu.sync_copy(acc_sh.at[pl.ds(r0, ZROWS)], acc1.at[pl.ds(r0, ZROWS)])

        @pl.when(s == 0)
        def _():
            pltpu.sync_copy(acc_sh.at[pl.ds(TAIL0, TAILN)], acc1.at[pl.ds(TAIL0, TAILN)])


def _run_sc(src, dst, dhbh_lo, dhbh_hi, eh_t, ce_lo, ce_hi):
    mesh = plsc.VectorSubcoreMesh(core_axis_name="c", subcore_axis_name="s",
                                  num_cores=NC, num_subcores=NS)
    fn = pl.kernel(
        _sc_body,
        out_type=[jax.ShapeDtypeStruct((E, H), F32),
                  jax.ShapeDtypeStruct((E, H), F32),
                  jax.ShapeDtypeStruct((N, D), F32),
                  jax.ShapeDtypeStruct((N, D), F32),
                  jax.ShapeDtypeStruct((NC, NS, 2, H), F32)],
        mesh=mesh,
        scratch_types=[pltpu.VMEM((CHUNK,), jnp.int32),
                       pltpu.VMEM((CHUNK,), jnp.int32),
                       pltpu.VMEM((CHUNK, D), F32),
                       pltpu.VMEM((CHUNK, D), F32),
                       pltpu.VMEM((CHUNK, H), F32),
                       pltpu.VMEM((CHUNK, H), F32),
                       pltpu.VMEM((CHUNK, D), F32),
                       pltpu.VMEM((ZROWS, D), F32),
                       pltpu.VMEM((2, H), F32),
                       pltpu.VMEM_SHARED((N, D), F32),
                       pltpu.SemaphoreType.DMA],
    )
    return fn(src, dst, dhbh_lo, dhbh_hi, eh_t, ce_lo, ce_hi)


def _fin_body(ah, a0, a1, st, gh, bh_, ge, be,
              hout, esc, esh):
    for hx, aref in enumerate((a0, a1)):
        sl = slice(hx * H, (hx + 1) * H)
        hn = ah[:, sl] + aref[:, :H] / (aref[:, H:] + 1e-6)
        mu = jnp.mean(hn, axis=0, keepdims=True)
        xc = hn - mu
        var = jnp.mean(xc * xc, axis=0, keepdims=True)
        y = gh[hx:hx + 1, :] * xc / jnp.sqrt(var + 1e-5) + bh_[hx:hx + 1, :]
        hout[:, sl] = jnp.maximum(y, 0.0)
    ssum = jnp.sum(st[...], axis=1)
    mu_e = ssum[:, 0, :] / E
    var_e = ssum[:, 1, :] / E - mu_e * mu_e
    scl = ge[...] / jnp.sqrt(var_e + 1e-5)
    esc[...] = scl
    esh[...] = be[...] - mu_e * scl


def _run_fin(ah, a0, a1, st, gh, bh_, ge, be):
    return pl.pallas_call(
        _fin_body,
        out_shape=[jax.ShapeDtypeStruct((N, D), F32),
                   jax.ShapeDtypeStruct((NC, H), F32),
                   jax.ShapeDtypeStruct((NC, H), F32)],
    )(ah, a0, a1, st, gh, bh_, ge, be)


def _eout_body(lo, hi, esc, esh, out):
    out[:, :H] = jnp.maximum(lo[...] * esc[0:1, :] + esh[0:1, :], 0.0)
    out[:, H:] = jnp.maximum(hi[...] * esc[1:2, :] + esh[1:2, :], 0.0)


def _run_eout(eij_lo, eij_hi, esc, esh):
    nb = 160
    rb = E // nb
    sspec = pl.BlockSpec((NC, H), lambda i: (0, 0))
    return pl.pallas_call(
        _eout_body,
        grid=(nb,),
        in_specs=[pl.BlockSpec((rb, H), lambda i: (i, 0)),
                  pl.BlockSpec((rb, H), lambda i: (i, 0)),
                  sspec, sspec],
        out_specs=pl.BlockSpec((rb, D), lambda i: (i, 0)),
        out_shape=jax.ShapeDtypeStruct((E, D), F32),
    )(eij_lo, eij_hi, esc, esh)


@jax.jit
def kernel(h, e, edge_index, A_w, A_b, B_w, B_b, C_w, C_b, D_w, D_b, E_w, E_b,
           bn_h_gamma, bn_h_beta, bn_e_gamma, bn_e_beta):
    src = edge_index[0]
    dst = edge_index[1]
    ah, dhbh_lo, dhbh_hi, eh_t = _run_proj(
        h, A_w, A_b.reshape(1, D), B_w, B_b.reshape(1, D),
        D_w, D_b.reshape(1, D), E_w, E_b.reshape(1, D))
    ce_lo, ce_hi = _run_ce(e, C_w, C_b.reshape(1, D))
    eij_lo, eij_hi, acc0, acc1, stats = _run_sc(
        src, dst, dhbh_lo, dhbh_hi, eh_t, ce_lo, ce_hi)
    h_out, esc, esh = _run_fin(
        ah, acc0, acc1, stats,
        bn_h_gamma.reshape(NC, H), bn_h_beta.reshape(NC, H),
        bn_e_gamma.reshape(NC, H), bn_e_beta.reshape(NC, H))
    e_out = _run_eout(eij_lo, eij_hi, esc, esh)
    return (h_out, e_out)
